# Initial kernel scaffold; baseline (speedup 1.0000x reference)
import jax
import jax.numpy as jnp
from jax.experimental import pallas as pl


def kernel(x1, edge_index1, x2, edge_index2, pos_anchor_edge_index, neg_anchor_edge_index, W1, b1, alpha, W2, b2):
    raise NotImplementedError("write your pallas kernel here")


# speedup vs baseline: 31.2858x; 31.2858x over previous
import functools

import jax
import jax.numpy as jnp
from jax import lax
from jax.experimental import pallas as pl
from jax.experimental.pallas import tpu as pltpu
from jax.experimental.pallas import tpu_sc as plsc

N = 10000
E = 320000
D = 128
NP = 10240
P = 4000
PA = 8192

NC = 2
NS = 16
NW = NC * NS
CH = 128
NCHUNK = E // CH
KMAX = -(-NCHUNK // NW)
TILE_NP = NP // NS
NACH = PA // CH

_SDS = jax.ShapeDtypeStruct
_f32 = jnp.float32
_i32 = jnp.int32


def _mesh():
    return plsc.VectorSubcoreMesh(
        core_axis_name="c", subcore_axis_name="s", num_cores=NC, num_subcores=NS
    )


_SC_PARAMS = pltpu.CompilerParams(use_tc_tiling_on_sc=False)


def _zero_fill(buf, n):
    z = jnp.zeros((16,), _f32)

    def body(i, _):
        buf[pl.ds(i * 16, 16)] = z
        return ()

    lax.fori_loop(0, n // 16, body, (), unroll=False)


@functools.partial(
    pl.kernel,
    out_type=(_SDS((NC * NP,), _f32), _SDS((NC * NP,), _f32)),
    mesh=_mesh(),
    compiler_params=_SC_PARAMS,
    scratch_types=[
        pltpu.VMEM((CH,), _i32),
        pltpu.VMEM((CH,), _f32),
        pltpu.VMEM((TILE_NP,), _f32),
        pltpu.VMEM_SHARED((NP,), _f32),
        pltpu.VMEM_SHARED((NP,), _f32),
    ],
)
def _deg_kernel(d1, d2, deg1_out, deg2_out,
                idx_v, ones_v, zbuf, acc1_sh, acc2_sh):
    cid = lax.axis_index("c")
    sid = lax.axis_index("s")
    wid = cid * NS + sid
    row0 = sid * TILE_NP

    one = jnp.ones((16,), _f32)

    def fill_ones(i, _):
        ones_v[pl.ds(i * 16, 16)] = one
        return ()

    lax.fori_loop(0, CH // 16, fill_ones, (), unroll=False)
    _zero_fill(zbuf, TILE_NP)
    pltpu.sync_copy(zbuf, acc1_sh.at[pl.ds(row0, TILE_NP)])
    pltpu.sync_copy(zbuf, acc2_sh.at[pl.ds(row0, TILE_NP)])
    plsc.subcore_barrier()

    def body(k, _):
        chunk = wid + NW * k

        @pl.when(chunk < NCHUNK)
        def _():
            off = chunk * CH
            pltpu.sync_copy(d1.at[pl.ds(off, CH)], idx_v)
            pltpu.sync_copy(ones_v, acc1_sh.at[idx_v], add=True)
            pltpu.sync_copy(d2.at[pl.ds(off, CH)], idx_v)
            pltpu.sync_copy(ones_v, acc2_sh.at[idx_v], add=True)
        return ()

    lax.fori_loop(0, KMAX, body, (), unroll=False)
    plsc.subcore_barrier()
    pltpu.sync_copy(acc1_sh.at[pl.ds(row0, TILE_NP)],
                    deg1_out.at[pl.ds(cid * NP + row0, TILE_NP)])
    pltpu.sync_copy(acc2_sh.at[pl.ds(row0, TILE_NP)],
                    deg2_out.at[pl.ds(cid * NP + row0, TILE_NP)])


@functools.partial(
    pl.kernel,
    out_type=tuple(_SDS((NC * NP,), _f32) for _ in range(4)),
    mesh=_mesh(),
    compiler_params=_SC_PARAMS,
    scratch_types=[
        pltpu.VMEM((CH,), _i32),
        pltpu.VMEM((CH,), _i32),
        pltpu.VMEM((CH,), _f32),
        pltpu.VMEM((CH,), _f32),
        pltpu.VMEM((TILE_NP,), _f32),
        pltpu.SemaphoreType.DMA,
        pltpu.VMEM_SHARED((NP,), _f32),
        pltpu.VMEM_SHARED((NP,), _f32),
        pltpu.VMEM_SHARED((NP,), _f32),
        pltpu.VMEM_SHARED((NP,), _f32),
        pltpu.VMEM_SHARED((NP,), _f32),
        pltpu.VMEM_SHARED((NP,), _f32),
        pltpu.VMEM_SHARED((NP,), _f32),
        pltpu.VMEM_SHARED((NP,), _f32),
    ],
)
def _conv_kernel(s1, d1, s2, d2, px1, py1, px2, py2,
                 ax1_out, ay1_out, ax2_out, ay2_out,
                 sidx_v, didx_v, gx_v, gy_v, zbuf, sem,
                 tx1_sh, ty1_sh, tx2_sh, ty2_sh,
                 ax1_sh, ay1_sh, ax2_sh, ay2_sh):
    cid = lax.axis_index("c")
    sid = lax.axis_index("s")
    wid = cid * NS + sid
    row0 = sid * TILE_NP
    sl = pl.ds(row0, TILE_NP)

    pltpu.sync_copy(px1.at[sl], tx1_sh.at[sl])
    pltpu.sync_copy(py1.at[sl], ty1_sh.at[sl])
    pltpu.sync_copy(px2.at[sl], tx2_sh.at[sl])
    pltpu.sync_copy(py2.at[sl], ty2_sh.at[sl])
    _zero_fill(zbuf, TILE_NP)
    pltpu.sync_copy(zbuf, ax1_sh.at[sl])
    pltpu.sync_copy(zbuf, ay1_sh.at[sl])
    pltpu.sync_copy(zbuf, ax2_sh.at[sl])
    pltpu.sync_copy(zbuf, ay2_sh.at[sl])
    plsc.subcore_barrier()

    def body(k, _):
        chunk = wid + NW * k

        @pl.when(chunk < NCHUNK)
        def _():
            off = chunk * CH
            for s, d, tx, ty, ax, ay in (
                (s1, d1, tx1_sh, ty1_sh, ax1_sh, ay1_sh),
                (s2, d2, tx2_sh, ty2_sh, ax2_sh, ay2_sh),
            ):
                pltpu.sync_copy(s.at[pl.ds(off, CH)], sidx_v)
                pltpu.sync_copy(d.at[pl.ds(off, CH)], didx_v)
                cg1 = pltpu.async_copy(tx.at[sidx_v], gx_v, sem)
                cg2 = pltpu.async_copy(ty.at[sidx_v], gy_v, sem)
                cg1.wait()
                cg2.wait()
                pltpu.sync_copy(gx_v, ax.at[didx_v], add=True)
                pltpu.sync_copy(gy_v, ay.at[didx_v], add=True)
        return ()

    lax.fori_loop(0, KMAX, body, (), unroll=False)
    plsc.subcore_barrier()
    osl = pl.ds(cid * NP + row0, TILE_NP)
    pltpu.sync_copy(ax1_sh.at[sl], ax1_out.at[osl])
    pltpu.sync_copy(ay1_sh.at[sl], ay1_out.at[osl])
    pltpu.sync_copy(ax2_sh.at[sl], ax2_out.at[osl])
    pltpu.sync_copy(ay2_sh.at[sl], ay2_out.at[osl])


@functools.partial(
    pl.kernel,
    out_type=tuple(_SDS((PA,), _f32) for _ in range(4)),
    mesh=_mesh(),
    compiler_params=_SC_PARAMS,
    scratch_types=[
        pltpu.VMEM((CH,), _i32),
        pltpu.VMEM((CH,), _f32),
        pltpu.VMEM((CH,), _f32),
        pltpu.SemaphoreType.DMA,
    ],
)
def _anchor_kernel(ex1, ey1, ex2, ey2, idx1, idx2,
                   sx1_out, sy1_out, sx2_out, sy2_out,
                   idx_v, gx_v, gy_v, sem):
    cid = lax.axis_index("c")
    sid = lax.axis_index("s")
    wid = cid * NS + sid

    def body(k, _):
        chunk = wid + NW * k

        @pl.when(chunk < NACH)
        def _():
            off = chunk * CH
            for ex, ey, idx, sx_out, sy_out in (
                (ex1, ey1, idx1, sx1_out, sy1_out),
                (ex2, ey2, idx2, sx2_out, sy2_out),
            ):
                pltpu.sync_copy(idx.at[pl.ds(off, CH)], idx_v)
                cg1 = pltpu.async_copy(ex.at[idx_v], gx_v, sem)
                cg2 = pltpu.async_copy(ey.at[idx_v], gy_v, sem)
                cg1.wait()
                cg2.wait()
                pltpu.sync_copy(gx_v, sx_out.at[pl.ds(off, CH)])
                pltpu.sync_copy(gy_v, sy_out.at[pl.ds(off, CH)])
        return ()

    lax.fori_loop(0, -(-NACH // NW), body, (), unroll=False)


_BN = 1024
_NB = NP // _BN


def _t1_body(x1_ref, x2_ref, w_ref, d1a_ref, d1b_ref, d2a_ref, d2b_ref,
             px1_ref, py1_ref, px2_ref, py2_ref, di1_ref, di2_ref):
    w = w_ref[...]
    for x_ref, da_ref, db_ref, px_ref, py_ref, di_ref in (
        (x1_ref, d1a_ref, d1b_ref, px1_ref, py1_ref, di1_ref),
        (x2_ref, d2a_ref, d2b_ref, px2_ref, py2_ref, di2_ref),
    ):
        h = lax.dot_general(w, x_ref[...], (((1,), (1,)), ((), ())),
                            preferred_element_type=_f32)
        deg = da_ref[...] + db_ref[...] + 1.0
        dinv = lax.rsqrt(deg)
        di_ref[...] = dinv
        px_ref[...] = h[0] * dinv
        py_ref[...] = h[1] * dinv


def _t1(x1p, x2p, w1t, degp1, degp2):
    blk = pl.BlockSpec((_BN,), lambda i: (i,))
    blk_hi = pl.BlockSpec((_BN,), lambda i: (i + _NB,))
    out = [_SDS((NP,), _f32) for _ in range(6)]
    return pl.pallas_call(
        _t1_body,
        grid=(_NB,),
        in_specs=[
            pl.BlockSpec((_BN, D), lambda i: (i, 0)),
            pl.BlockSpec((_BN, D), lambda i: (i, 0)),
            pl.BlockSpec((8, D), lambda i: (0, 0)),
            blk, blk_hi, blk, blk_hi,
        ],
        out_specs=[blk] * 6,
        out_shape=out,
    )(x1p, x2p, w1t, degp1, degp1, degp2, degp2)


def _t2_body(ax1a, ax1b, ay1a, ay1b, ax2a, ax2b, ay2a, ay2b,
             px1, py1, px2, py2, di1, di2, w2, alpha, b1,
             qx1, qy1, qx2, qy2):
    w00 = w2[0, 0]
    w01 = w2[0, 1]
    w10 = w2[1, 0]
    w11 = w2[1, 1]
    a0 = alpha[0]
    a1 = alpha[1]
    b10 = b1[0]
    b11 = b1[1]
    for axa, axb, aya, ayb, px, py, di, qx, qy in (
        (ax1a, ax1b, ay1a, ay1b, px1, py1, di1, qx1, qy1),
        (ax2a, ax2b, ay2a, ay2b, px2, py2, di2, qx2, qy2),
    ):
        dinv = di[...]
        hx = dinv * (axa[...] + axb[...] + px[...]) + b10
        hy = dinv * (aya[...] + ayb[...] + py[...]) + b11
        gx = jnp.where(hx >= 0.0, hx, a0 * hx)
        gy = jnp.where(hy >= 0.0, hy, a1 * hy)
        qx[...] = dinv * (gx * w00 + gy * w10)
        qy[...] = dinv * (gx * w01 + gy * w11)


def _t2(acc1, acc2, p1, p2, dinv1, dinv2, w2, alpha, b1):
    blk = pl.BlockSpec((_BN,), lambda i: (i,))
    blk_hi = pl.BlockSpec((_BN,), lambda i: (i + _NB,))
    smem = pl.BlockSpec(memory_space=pltpu.SMEM)
    accs = []
    for a in (*acc1, *acc2):
        accs.extend([a, a])
    return pl.pallas_call(
        _t2_body,
        grid=(_NB,),
        in_specs=[blk, blk_hi] * 4 + [blk] * 6 + [smem] * 3,
        out_specs=[blk] * 4,
        out_shape=[_SDS((NP,), _f32) for _ in range(4)],
    )(*accs, *p1, *p2, dinv1, dinv2, w2, alpha, b1)


def _t3_body(ax1a, ax1b, ay1a, ay1b, ax2a, ax2b, ay2a, ay2b,
             qx1, qy1, qx2, qy2, di1, di2, b2,
             ex1, ey1, ex2, ey2):
    b20 = b2[0]
    b21 = b2[1]
    for axa, axb, aya, ayb, qx, qy, di, ex, ey in (
        (ax1a, ax1b, ay1a, ay1b, qx1, qy1, di1, ex1, ey1),
        (ax2a, ax2b, ay2a, ay2b, qx2, qy2, di2, ex2, ey2),
    ):
        dinv = di[...]
        ex[...] = dinv * (axa[...] + axb[...] + qx[...]) + b20
        ey[...] = dinv * (aya[...] + ayb[...] + qy[...]) + b21


def _t3(acc1, acc2, q1, q2, dinv1, dinv2, b2):
    blk = pl.BlockSpec((_BN,), lambda i: (i,))
    blk_hi = pl.BlockSpec((_BN,), lambda i: (i + _NB,))
    smem = pl.BlockSpec(memory_space=pltpu.SMEM)
    accs = []
    for a in (*acc1, *acc2):
        accs.extend([a, a])
    return pl.pallas_call(
        _t3_body,
        grid=(_NB,),
        in_specs=[blk, blk_hi] * 4 + [blk] * 6 + [smem],
        out_specs=[blk] * 4,
        out_shape=[_SDS((NP,), _f32) for _ in range(4)],
    )(*accs, *q1, *q2, dinv1, dinv2, b2)


def _t4_body(sx1, sy1, sx2, sy2, ox1, oy1, ox2, oy2):
    for sx, sy, ox, oy in ((sx1, sy1, ox1, oy1), (sx2, sy2, ox2, oy2)):
        x = sx[...]
        y = sy[...]
        s = lax.rsqrt(jnp.maximum(x * x + y * y, 1e-24))
        ox[...] = x * s
        oy[...] = y * s


def _t4(sx1, sy1, sx2, sy2):
    spec = pl.BlockSpec((PA,), lambda: (0,))
    return pl.pallas_call(
        _t4_body,
        in_specs=[spec] * 4,
        out_specs=[spec] * 4,
        out_shape=[_SDS((PA,), _f32) for _ in range(4)],
    )(sx1, sy1, sx2, sy2)


def kernel(x1, edge_index1, x2, edge_index2, pos_anchor_edge_index,
           neg_anchor_edge_index, W1, b1, alpha, W2, b2):
    e1 = edge_index1.astype(_i32)
    e2 = edge_index2.astype(_i32)
    s1, d1 = e1[0], e1[1]
    s2, d2 = e2[0], e2[1]
    x1p = jnp.pad(x1.astype(_f32), ((0, NP - N), (0, 0)))
    x2p = jnp.pad(x2.astype(_f32), ((0, NP - N), (0, 0)))
    w1t = jnp.zeros((8, D), _f32).at[0:2, :].set(W1.astype(_f32).T)

    pad_idx = (jnp.arange(PA - 2 * P, dtype=_i32) * 37) % N
    t1_idx = jnp.concatenate([pos_anchor_edge_index[0].astype(_i32),
                              neg_anchor_edge_index[0].astype(_i32), pad_idx])
    t2_idx = jnp.concatenate([pos_anchor_edge_index[1].astype(_i32),
                              neg_anchor_edge_index[1].astype(_i32), pad_idx])

    degp1, degp2 = _deg_kernel(d1, d2)
    px1, py1, px2, py2, dinv1, dinv2 = _t1(x1p, x2p, w1t, degp1, degp2)
    acc = _conv_kernel(s1, d1, s2, d2, px1, py1, px2, py2)
    qx1, qy1, qx2, qy2 = _t2(acc[0:2], acc[2:4], (px1, py1), (px2, py2),
                             dinv1, dinv2, W2.astype(_f32),
                             alpha.astype(_f32), b1.astype(_f32))
    acc2 = _conv_kernel(s1, d1, s2, d2, qx1, qy1, qx2, qy2)
    ex1, ey1, ex2, ey2 = _t3(acc2[0:2], acc2[2:4], (qx1, qy1), (qx2, qy2),
                             dinv1, dinv2, b2.astype(_f32))
    sx1, sy1, sx2, sy2 = _anchor_kernel(ex1, ey1, ex2, ey2, t1_idx, t2_idx)
    ox1, oy1, ox2, oy2 = _t4(sx1, sy1, sx2, sy2)
    o1 = jnp.stack([ox1[: 2 * P], oy1[: 2 * P]], axis=-1)
    o2 = jnp.stack([ox2[: 2 * P], oy2[: 2 * P]], axis=-1)
    return o1, o2


# speedup vs baseline: 122.3083x; 3.9094x over previous
import functools

import jax
import jax.numpy as jnp
from jax import lax
from jax.experimental import pallas as pl
from jax.experimental.pallas import tpu as pltpu
from jax.experimental.pallas import tpu_sc as plsc

N = 10000
E = 320000
D = 128
NP = 10240
P = 4000
PA = 8192

NC = 2
NS = 16
NW = NC * NS
CH = 2560
ACH = 512
NCHUNK = E // CH
KMAX = -(-NCHUNK // NW)
TILE_NP = NP // NS
NACH = PA // ACH

_SDS = jax.ShapeDtypeStruct
_f32 = jnp.float32
_i32 = jnp.int32


def _mesh():
    return plsc.VectorSubcoreMesh(
        core_axis_name="c", subcore_axis_name="s", num_cores=NC, num_subcores=NS
    )


_SC_PARAMS = pltpu.CompilerParams(use_tc_tiling_on_sc=False)


def _zero_fill(buf, n):
    z = jnp.zeros((16,), _f32)

    def body(i, _):
        buf[pl.ds(i * 16, 16)] = z
        return ()

    lax.fori_loop(0, n // 16, body, (), unroll=False)


@functools.partial(
    pl.kernel,
    out_type=(_SDS((NC * NP,), _f32), _SDS((NC * NP,), _f32)),
    mesh=_mesh(),
    compiler_params=_SC_PARAMS,
    scratch_types=[
        pltpu.VMEM((2, CH), _i32),
        pltpu.VMEM((2, CH), _i32),
        pltpu.VMEM((CH,), _f32),
        pltpu.VMEM((TILE_NP,), _f32),
        pltpu.SemaphoreType.DMA,
        pltpu.SemaphoreType.DMA,
        pltpu.VMEM_SHARED((NP,), _f32),
        pltpu.VMEM_SHARED((NP,), _f32),
    ],
)
def _deg_kernel(d1, d2, deg1_out, deg2_out,
                d1b, d2b, ones_v, zbuf, isem, ssem, acc1_sh, acc2_sh):
    cid = lax.axis_index("c")
    sid = lax.axis_index("s")
    wid = cid * NS + sid
    row0 = sid * TILE_NP

    one = jnp.ones((16,), _f32)

    def fill_ones(i, _):
        ones_v[pl.ds(i * 16, 16)] = one
        return ()

    lax.fori_loop(0, CH // 16, fill_ones, (), unroll=False)
    _zero_fill(zbuf, TILE_NP)
    pltpu.sync_copy(zbuf, acc1_sh.at[pl.ds(row0, TILE_NP)])
    pltpu.sync_copy(zbuf, acc2_sh.at[pl.ds(row0, TILE_NP)])
    plsc.subcore_barrier()

    def issue_idx(k, b):
        chunk = wid + NW * k

        @pl.when(chunk < NCHUNK)
        def _():
            off = chunk * CH
            pltpu.async_copy(d1.at[pl.ds(off, CH)], d1b.at[b], isem)
            pltpu.async_copy(d2.at[pl.ds(off, CH)], d2b.at[b], isem)

    issue_idx(0, 0)

    def wait_scatters(b):
        pltpu.make_async_copy(ones_v, acc1_sh.at[d1b.at[b]], ssem).wait()
        pltpu.make_async_copy(ones_v, acc2_sh.at[d2b.at[b]], ssem).wait()

    def body(k2, _):
        for b in (0, 1):
            k = 2 * k2 + b
            chunk = wid + NW * k

            @pl.when(chunk < NCHUNK)
            def _():
                off = chunk * CH
                pltpu.make_async_copy(d1.at[pl.ds(off, CH)], d1b.at[b],
                                      isem).wait()
                pltpu.make_async_copy(d2.at[pl.ds(off, CH)], d2b.at[b],
                                      isem).wait()

                @pl.when(k >= 1)
                def _w():
                    wait_scatters(1 - b)

                issue_idx(k + 1, 1 - b)
                pltpu.async_copy(ones_v, acc1_sh.at[d1b.at[b]], ssem,
                                 add=True)
                pltpu.async_copy(ones_v, acc2_sh.at[d2b.at[b]], ssem,
                                 add=True)
        return ()

    lax.fori_loop(0, -(-KMAX // 2), body, (), unroll=False)
    wait_scatters(0)
    plsc.subcore_barrier()
    pltpu.sync_copy(acc1_sh.at[pl.ds(row0, TILE_NP)],
                    deg1_out.at[pl.ds(cid * NP + row0, TILE_NP)])
    pltpu.sync_copy(acc2_sh.at[pl.ds(row0, TILE_NP)],
                    deg2_out.at[pl.ds(cid * NP + row0, TILE_NP)])


@functools.partial(
    pl.kernel,
    out_type=tuple(_SDS((NC * NP,), _f32) for _ in range(4)),
    mesh=_mesh(),
    compiler_params=_SC_PARAMS,
    scratch_types=[
        pltpu.VMEM((2, CH), _i32),
        pltpu.VMEM((2, CH), _i32),
        pltpu.VMEM((2, CH), _i32),
        pltpu.VMEM((2, CH), _i32),
        pltpu.VMEM((2, CH), _f32),
        pltpu.VMEM((2, CH), _f32),
        pltpu.VMEM((2, CH), _f32),
        pltpu.VMEM((2, CH), _f32),
        pltpu.VMEM((TILE_NP,), _f32),
        pltpu.SemaphoreType.DMA,
        pltpu.SemaphoreType.DMA,
        pltpu.SemaphoreType.DMA,
        pltpu.VMEM_SHARED((NP,), _f32),
        pltpu.VMEM_SHARED((NP,), _f32),
        pltpu.VMEM_SHARED((NP,), _f32),
        pltpu.VMEM_SHARED((NP,), _f32),
        pltpu.VMEM_SHARED((NP,), _f32),
        pltpu.VMEM_SHARED((NP,), _f32),
        pltpu.VMEM_SHARED((NP,), _f32),
        pltpu.VMEM_SHARED((NP,), _f32),
    ],
)
def _conv_kernel(s1, d1, s2, d2, px1, py1, px2, py2,
                 ax1_out, ay1_out, ax2_out, ay2_out,
                 s1b, d1b, s2b, d2b, gx1_v, gy1_v, gx2_v, gy2_v, zbuf,
                 isem, gsem, ssem,
                 tx1_sh, ty1_sh, tx2_sh, ty2_sh,
                 ax1_sh, ay1_sh, ax2_sh, ay2_sh):
    cid = lax.axis_index("c")
    sid = lax.axis_index("s")
    wid = cid * NS + sid
    row0 = sid * TILE_NP
    sl = pl.ds(row0, TILE_NP)

    pltpu.sync_copy(px1.at[sl], tx1_sh.at[sl])
    pltpu.sync_copy(py1.at[sl], ty1_sh.at[sl])
    pltpu.sync_copy(px2.at[sl], tx2_sh.at[sl])
    pltpu.sync_copy(py2.at[sl], ty2_sh.at[sl])
    _zero_fill(zbuf, TILE_NP)
    pltpu.sync_copy(zbuf, ax1_sh.at[sl])
    pltpu.sync_copy(zbuf, ay1_sh.at[sl])
    pltpu.sync_copy(zbuf, ax2_sh.at[sl])
    pltpu.sync_copy(zbuf, ay2_sh.at[sl])
    plsc.subcore_barrier()

    def issue_idx(k, b):
        chunk = wid + NW * k

        @pl.when(chunk < NCHUNK)
        def _():
            off = chunk * CH
            pltpu.async_copy(s1.at[pl.ds(off, CH)], s1b.at[b], isem)
            pltpu.async_copy(d1.at[pl.ds(off, CH)], d1b.at[b], isem)
            pltpu.async_copy(s2.at[pl.ds(off, CH)], s2b.at[b], isem)
            pltpu.async_copy(d2.at[pl.ds(off, CH)], d2b.at[b], isem)

    issue_idx(0, 0)

    def wait_scatters(b):
        for gv, axsh, db in ((gx1_v, ax1_sh, d1b), (gy1_v, ay1_sh, d1b),
                             (gx2_v, ax2_sh, d2b), (gy2_v, ay2_sh, d2b)):
            pltpu.make_async_copy(gv.at[b], axsh.at[db.at[b]], ssem).wait()

    def body(k2, _):
        for b in (0, 1):
            k = 2 * k2 + b
            chunk = wid + NW * k

            @pl.when(chunk < NCHUNK)
            def _():
                off = chunk * CH
                for s, d, sb, db in ((s1, d1, s1b, d1b), (s2, d2, s2b, d2b)):
                    pltpu.make_async_copy(s.at[pl.ds(off, CH)], sb.at[b],
                                          isem).wait()
                    pltpu.make_async_copy(d.at[pl.ds(off, CH)], db.at[b],
                                          isem).wait()
                cgs = [
                    pltpu.async_copy(tx1_sh.at[s1b.at[b]], gx1_v.at[b], gsem),
                    pltpu.async_copy(ty1_sh.at[s1b.at[b]], gy1_v.at[b], gsem),
                    pltpu.async_copy(tx2_sh.at[s2b.at[b]], gx2_v.at[b], gsem),
                    pltpu.async_copy(ty2_sh.at[s2b.at[b]], gy2_v.at[b], gsem),
                ]

                @pl.when(k >= 1)
                def _w():
                    wait_scatters(1 - b)

                issue_idx(k + 1, 1 - b)
                for cg in cgs:
                    cg.wait()
                pltpu.async_copy(gx1_v.at[b], ax1_sh.at[d1b.at[b]], ssem,
                                 add=True)
                pltpu.async_copy(gy1_v.at[b], ay1_sh.at[d1b.at[b]], ssem,
                                 add=True)
                pltpu.async_copy(gx2_v.at[b], ax2_sh.at[d2b.at[b]], ssem,
                                 add=True)
                pltpu.async_copy(gy2_v.at[b], ay2_sh.at[d2b.at[b]], ssem,
                                 add=True)
        return ()

    lax.fori_loop(0, -(-KMAX // 2), body, (), unroll=False)
    wait_scatters(0)
    plsc.subcore_barrier()
    osl = pl.ds(cid * NP + row0, TILE_NP)
    pltpu.sync_copy(ax1_sh.at[sl], ax1_out.at[osl])
    pltpu.sync_copy(ay1_sh.at[sl], ay1_out.at[osl])
    pltpu.sync_copy(ax2_sh.at[sl], ax2_out.at[osl])
    pltpu.sync_copy(ay2_sh.at[sl], ay2_out.at[osl])


@functools.partial(
    pl.kernel,
    out_type=tuple(_SDS((PA,), _f32) for _ in range(4)),
    mesh=_mesh(),
    compiler_params=_SC_PARAMS,
    scratch_types=[
        pltpu.VMEM((2, ACH), _i32),
        pltpu.VMEM((2, ACH), _i32),
        pltpu.VMEM((ACH,), _f32),
        pltpu.VMEM((ACH,), _f32),
        pltpu.VMEM((ACH,), _f32),
        pltpu.VMEM((ACH,), _f32),
        pltpu.SemaphoreType.DMA,
        pltpu.SemaphoreType.DMA,
        pltpu.SemaphoreType.DMA,
    ],
)
def _anchor_kernel(ex1, ey1, ex2, ey2, idx1, idx2,
                   sx1_out, sy1_out, sx2_out, sy2_out,
                   i1b, i2b, gx1_v, gy1_v, gx2_v, gy2_v, isem, gsem, osem):
    cid = lax.axis_index("c")
    sid = lax.axis_index("s")
    wid = cid * NS + sid

    def issue_idx(k, b):
        chunk = wid + NW * k

        @pl.when(chunk < NACH)
        def _():
            off = chunk * ACH
            pltpu.async_copy(idx1.at[pl.ds(off, ACH)], i1b.at[b], isem)
            pltpu.async_copy(idx2.at[pl.ds(off, ACH)], i2b.at[b], isem)

    issue_idx(0, 0)
    issue_idx(1, 1)

    def body(k, b):
        chunk = wid + NW * k

        @pl.when(chunk < NACH)
        def _():
            off = chunk * ACH
            pltpu.make_async_copy(idx1.at[pl.ds(off, ACH)], i1b.at[b],
                                  isem).wait()
            pltpu.make_async_copy(idx2.at[pl.ds(off, ACH)], i2b.at[b],
                                  isem).wait()
            cgs = [
                pltpu.async_copy(ex1.at[i1b.at[b]], gx1_v, gsem),
                pltpu.async_copy(ey1.at[i1b.at[b]], gy1_v, gsem),
                pltpu.async_copy(ex2.at[i2b.at[b]], gx2_v, gsem),
                pltpu.async_copy(ey2.at[i2b.at[b]], gy2_v, gsem),
            ]
            for cg in cgs:
                cg.wait()
            cos = [
                pltpu.async_copy(gx1_v, sx1_out.at[pl.ds(off, ACH)], osem),
                pltpu.async_copy(gy1_v, sy1_out.at[pl.ds(off, ACH)], osem),
                pltpu.async_copy(gx2_v, sx2_out.at[pl.ds(off, ACH)], osem),
                pltpu.async_copy(gy2_v, sy2_out.at[pl.ds(off, ACH)], osem),
            ]
            for co in cos:
                co.wait()

    body(0, 0)
    body(1, 1)


_BN = 1024
_NB = NP // _BN


def _t1_body(x1_ref, x2_ref, w_ref, d1a_ref, d1b_ref, d2a_ref, d2b_ref,
             px1_ref, py1_ref, px2_ref, py2_ref, di1_ref, di2_ref):
    w = w_ref[...]
    for x_ref, da_ref, db_ref, px_ref, py_ref, di_ref in (
        (x1_ref, d1a_ref, d1b_ref, px1_ref, py1_ref, di1_ref),
        (x2_ref, d2a_ref, d2b_ref, px2_ref, py2_ref, di2_ref),
    ):
        h = lax.dot_general(w, x_ref[...], (((1,), (1,)), ((), ())),
                            preferred_element_type=_f32)
        deg = da_ref[...] + db_ref[...] + 1.0
        dinv = lax.rsqrt(deg)
        di_ref[...] = dinv
        px_ref[...] = h[0] * dinv
        py_ref[...] = h[1] * dinv


def _t1(x1p, x2p, w1t, degp1, degp2):
    blk = pl.BlockSpec((_BN,), lambda i: (i,))
    blk_hi = pl.BlockSpec((_BN,), lambda i: (i + _NB,))
    out = [_SDS((NP,), _f32) for _ in range(6)]
    return pl.pallas_call(
        _t1_body,
        grid=(_NB,),
        in_specs=[
            pl.BlockSpec((_BN, D), lambda i: (i, 0)),
            pl.BlockSpec((_BN, D), lambda i: (i, 0)),
            pl.BlockSpec((8, D), lambda i: (0, 0)),
            blk, blk_hi, blk, blk_hi,
        ],
        out_specs=[blk] * 6,
        out_shape=out,
    )(x1p, x2p, w1t, degp1, degp1, degp2, degp2)


def _t2_body(ax1a, ax1b, ay1a, ay1b, ax2a, ax2b, ay2a, ay2b,
             px1, py1, px2, py2, di1, di2, w2, alpha, b1,
             qx1, qy1, qx2, qy2):
    w00 = w2[0, 0]
    w01 = w2[0, 1]
    w10 = w2[1, 0]
    w11 = w2[1, 1]
    a0 = alpha[0]
    a1 = alpha[1]
    b10 = b1[0]
    b11 = b1[1]
    for axa, axb, aya, ayb, px, py, di, qx, qy in (
        (ax1a, ax1b, ay1a, ay1b, px1, py1, di1, qx1, qy1),
        (ax2a, ax2b, ay2a, ay2b, px2, py2, di2, qx2, qy2),
    ):
        dinv = di[...]
        hx = dinv * (axa[...] + axb[...] + px[...]) + b10
        hy = dinv * (aya[...] + ayb[...] + py[...]) + b11
        gx = jnp.where(hx >= 0.0, hx, a0 * hx)
        gy = jnp.where(hy >= 0.0, hy, a1 * hy)
        qx[...] = dinv * (gx * w00 + gy * w10)
        qy[...] = dinv * (gx * w01 + gy * w11)


def _t2(acc1, acc2, p1, p2, dinv1, dinv2, w2, alpha, b1):
    blk = pl.BlockSpec((_BN,), lambda i: (i,))
    blk_hi = pl.BlockSpec((_BN,), lambda i: (i + _NB,))
    smem = pl.BlockSpec(memory_space=pltpu.SMEM)
    accs = []
    for a in (*acc1, *acc2):
        accs.extend([a, a])
    return pl.pallas_call(
        _t2_body,
        grid=(_NB,),
        in_specs=[blk, blk_hi] * 4 + [blk] * 6 + [smem] * 3,
        out_specs=[blk] * 4,
        out_shape=[_SDS((NP,), _f32) for _ in range(4)],
    )(*accs, *p1, *p2, dinv1, dinv2, w2, alpha, b1)


def _t3_body(ax1a, ax1b, ay1a, ay1b, ax2a, ax2b, ay2a, ay2b,
             qx1, qy1, qx2, qy2, di1, di2, b2,
             ex1, ey1, ex2, ey2):
    b20 = b2[0]
    b21 = b2[1]
    for axa, axb, aya, ayb, qx, qy, di, ex, ey in (
        (ax1a, ax1b, ay1a, ay1b, qx1, qy1, di1, ex1, ey1),
        (ax2a, ax2b, ay2a, ay2b, qx2, qy2, di2, ex2, ey2),
    ):
        dinv = di[...]
        ex[...] = dinv * (axa[...] + axb[...] + qx[...]) + b20
        ey[...] = dinv * (aya[...] + ayb[...] + qy[...]) + b21


def _t3(acc1, acc2, q1, q2, dinv1, dinv2, b2):
    blk = pl.BlockSpec((_BN,), lambda i: (i,))
    blk_hi = pl.BlockSpec((_BN,), lambda i: (i + _NB,))
    smem = pl.BlockSpec(memory_space=pltpu.SMEM)
    accs = []
    for a in (*acc1, *acc2):
        accs.extend([a, a])
    return pl.pallas_call(
        _t3_body,
        grid=(_NB,),
        in_specs=[blk, blk_hi] * 4 + [blk] * 6 + [smem],
        out_specs=[blk] * 4,
        out_shape=[_SDS((NP,), _f32) for _ in range(4)],
    )(*accs, *q1, *q2, dinv1, dinv2, b2)


def _t4_body(sx1, sy1, sx2, sy2, ox1, oy1, ox2, oy2):
    for sx, sy, ox, oy in ((sx1, sy1, ox1, oy1), (sx2, sy2, ox2, oy2)):
        x = sx[...]
        y = sy[...]
        s = lax.rsqrt(jnp.maximum(x * x + y * y, 1e-24))
        ox[...] = x * s
        oy[...] = y * s


def _t4(sx1, sy1, sx2, sy2):
    spec = pl.BlockSpec((PA,), lambda: (0,))
    return pl.pallas_call(
        _t4_body,
        in_specs=[spec] * 4,
        out_specs=[spec] * 4,
        out_shape=[_SDS((PA,), _f32) for _ in range(4)],
    )(sx1, sy1, sx2, sy2)


def kernel(x1, edge_index1, x2, edge_index2, pos_anchor_edge_index,
           neg_anchor_edge_index, W1, b1, alpha, W2, b2):
    e1 = edge_index1.astype(_i32)
    e2 = edge_index2.astype(_i32)
    s1, d1 = e1[0], e1[1]
    s2, d2 = e2[0], e2[1]
    x1p = x1.astype(_f32)
    x2p = x2.astype(_f32)
    w1t = jnp.zeros((8, D), _f32).at[0:2, :].set(W1.astype(_f32).T)

    pad_idx = (jnp.arange(PA - 2 * P, dtype=_i32) * 37) % N
    t1_idx = jnp.concatenate([pos_anchor_edge_index[0].astype(_i32),
                              neg_anchor_edge_index[0].astype(_i32), pad_idx])
    t2_idx = jnp.concatenate([pos_anchor_edge_index[1].astype(_i32),
                              neg_anchor_edge_index[1].astype(_i32), pad_idx])

    degp1, degp2 = _deg_kernel(d1, d2)
    px1, py1, px2, py2, dinv1, dinv2 = _t1(x1p, x2p, w1t, degp1, degp2)
    acc = _conv_kernel(s1, d1, s2, d2, px1, py1, px2, py2)
    qx1, qy1, qx2, qy2 = _t2(acc[0:2], acc[2:4], (px1, py1), (px2, py2),
                             dinv1, dinv2, W2.astype(_f32),
                             alpha.astype(_f32), b1.astype(_f32))
    acc2 = _conv_kernel(s1, d1, s2, d2, qx1, qy1, qx2, qy2)
    ex1, ey1, ex2, ey2 = _t3(acc2[0:2], acc2[2:4], (qx1, qy1), (qx2, qy2),
                             dinv1, dinv2, b2.astype(_f32))
    sx1, sy1, sx2, sy2 = _anchor_kernel(ex1, ey1, ex2, ey2, t1_idx, t2_idx)
    ox1, oy1, ox2, oy2 = _t4(sx1, sy1, sx2, sy2)
    o1 = jnp.stack([ox1[: 2 * P], oy1[: 2 * P]], axis=-1)
    o2 = jnp.stack([ox2[: 2 * P], oy2[: 2 * P]], axis=-1)
    return o1, o2
